# baseline (speedup 1.0000x reference)
import jax
import jax.numpy as jnp
from jax.experimental import pallas as pl
from jax.experimental.pallas import tpu as pltpu

B2, L, H, D = 16, 2048, 16, 128
BL = 512


def _copy_scatter_body(idx_ref, src_ref, row_ref, out_ref):
    l = pl.program_id(1)
    out_ref[...] = src_ref[...]
    r = idx_ref[0] - l * BL
    @pl.when(jnp.logical_and(r >= 0, r < BL))
    def _():
        out_ref[0, pl.ds(r, 1)] = row_ref[0]


def _one_cache(cache, row, idx):
    return pl.pallas_call(
        _copy_scatter_body,
        grid=(B2, L // BL),
        in_specs=[
            pl.BlockSpec(memory_space=pltpu.MemorySpace.SMEM),
            pl.BlockSpec((1, BL, H, D), lambda b, l: (b, l, 0, 0)),
            pl.BlockSpec((1, 1, H, D), lambda b, l: (b, 0, 0, 0)),
        ],
        out_specs=pl.BlockSpec((1, BL, H, D), lambda b, l: (b, l, 0, 0)),
        out_shape=jax.ShapeDtypeStruct((B2, L, H, D), jnp.float32),
    )(idx, cache, row)


def kernel(cache_k, cache_v, k, v, current_idx):
    idx = jnp.asarray(current_idx, jnp.int32).reshape(1)
    ok = _one_cache(cache_k, k, idx)
    ov = _one_cache(cache_v, v, idx)
    return ok, ov


# speedup vs baseline: 1.0151x; 1.0151x over previous
import jax
import jax.numpy as jnp
from jax.experimental import pallas as pl
from jax.experimental.pallas import tpu as pltpu

B2, L, H, D = 16, 2048, 16, 128
BL = 512


def _copy_scatter_body(idx_ref, ck_ref, cv_ref, k_ref, v_ref, ok_ref, ov_ref):
    l = pl.program_id(1)
    ok_ref[...] = ck_ref[...]
    ov_ref[...] = cv_ref[...]
    r = idx_ref[0] - l * BL
    @pl.when(jnp.logical_and(r >= 0, r < BL))
    def _():
        ok_ref[0, pl.ds(r, 1)] = k_ref[0]
        ov_ref[0, pl.ds(r, 1)] = v_ref[0]


def kernel(cache_k, cache_v, k, v, current_idx):
    idx = jnp.asarray(current_idx, jnp.int32).reshape(1)
    blk = pl.BlockSpec((1, BL, H, D), lambda b, l: (b, l, 0, 0))
    rowblk = pl.BlockSpec((1, 1, H, D), lambda b, l: (b, 0, 0, 0))
    ok, ov = pl.pallas_call(
        _copy_scatter_body,
        grid=(B2, L // BL),
        in_specs=[
            pl.BlockSpec(memory_space=pltpu.MemorySpace.SMEM),
            blk, blk, rowblk, rowblk,
        ],
        out_specs=[blk, blk],
        out_shape=[
            jax.ShapeDtypeStruct((B2, L, H, D), jnp.float32),
            jax.ShapeDtypeStruct((B2, L, H, D), jnp.float32),
        ],
    )(idx, cache_k, cache_v, k, v)
    return ok, ov
